# baseline (speedup 1.0000x reference)
import jax
import jax.numpy as jnp
from jax import lax
from jax.experimental import pallas as pl
from jax.experimental.pallas import tpu as pltpu
from jax.experimental.pallas import tpu_sc as plsc

NC = 2
NS = 16
LANES = 16

CHUNK = 128
D = 64
DH = 32


def _sc_kernel_body(widx_hbm, pidx_hbm, wa_hbm, wb_hbm, pos_hbm, out_hbm,
                    wi_v, pi_v, abuf0, abuf1, bbuf0, bbuf1, pbuf0, pbuf1,
                    sema0, sema1, semb0, semb1, semp0, semp1, sems0, sems1):
    wid = lax.axis_index("c") * NS + lax.axis_index("s")
    n_per_tile = widx_hbm.shape[0] // (NC * NS)
    n_chunks = n_per_tile // CHUNK
    base = wid * n_per_tile

    pltpu.sync_copy(widx_hbm.at[pl.ds(base, n_per_tile)], wi_v)
    pltpu.sync_copy(pidx_hbm.at[pl.ds(base, n_per_tile)], pi_v)

    abufs = [abuf0, abuf1]
    bbufs = [bbuf0, bbuf1]
    pbufs = [pbuf0, pbuf1]
    semas = [sema0, sema1]
    sembs = [semb0, semb1]
    semps = [semp0, semp1]
    semss = [sems0, sems1]
    cs = [None, None]

    def issue_chunk(k, b):
        koff = k * CHUNK
        abuf, bbuf, pbuf = abufs[b], bbufs[b], pbufs[b]
        sema, semb, semp = semas[b], sembs[b], semps[b]

        def issue_body(g, _):
            iw = wi_v[pl.ds(koff + g * LANES, LANES)]
            ip = pi_v[pl.ds(koff + g * LANES, LANES)]
            for j in range(LANES):
                r = g * LANES + j
                pltpu.async_copy(wa_hbm.at[iw[j]], abuf.at[r], sema)
                pltpu.async_copy(wb_hbm.at[iw[j]], bbuf.at[r], semb)
                pltpu.async_copy(pos_hbm.at[ip[j]], pbuf.at[r], semp)
            return 0

        lax.fori_loop(0, CHUNK // LANES, issue_body, 0, unroll=False)

    def drain_chunk(b):
        abuf, bbuf, pbuf = abufs[b], bbufs[b], pbufs[b]
        sema, semb, semp = semas[b], sembs[b], semps[b]

        def drain_body(r, _):
            pltpu.make_async_copy(wa_hbm.at[0], abuf.at[r], sema).wait()
            pltpu.make_async_copy(wb_hbm.at[0], bbuf.at[r], semb).wait()
            pltpu.make_async_copy(pos_hbm.at[0], pbuf.at[r], semp).wait()
            return 0

        lax.fori_loop(0, CHUNK, drain_body, 0, unroll=False)

    issue_chunk(0, 0)

    for k in range(n_chunks):
        b = k % 2
        nb = (k + 1) % 2
        if k + 1 < n_chunks:
            if cs[nb] is not None:
                cs[nb].wait()
            issue_chunk(k + 1, nb)
        drain_chunk(b)

        abuf, bbuf, pbuf = abufs[b], bbufs[b], pbufs[b]

        def add_body(r, _):
            for c in range(DH // LANES):
                slh = pl.ds(c * LANES, LANES)
                pbuf[r, slh] = pbuf[r, slh] + abuf[r, slh]
                slo = pl.ds(DH + c * LANES, LANES)
                pbuf[r, slo] = pbuf[r, slo] + bbuf[r, slh]
            return 0

        lax.fori_loop(0, CHUNK, add_body, 0, unroll=False)
        cs[b] = pltpu.async_copy(
            pbuf, out_hbm.at[pl.ds(base + k * CHUNK, CHUNK)], semss[b])

    for c in cs:
        if c is not None:
            c.wait()


def kernel(src_word, src_pos, word_emb, pos_emb):
    B, S = src_word.shape
    n = B * S
    widx = src_word.reshape(n)
    pidx = src_pos.reshape(n)
    wa = word_emb[:, :DH]
    wb = word_emb[:, DH:]

    mesh = plsc.VectorSubcoreMesh(core_axis_name="c", subcore_axis_name="s",
                                  num_cores=NC, num_subcores=NS)
    n_per_tile = n // (NC * NS)
    run = pl.kernel(
        _sc_kernel_body,
        out_type=jax.ShapeDtypeStruct((n, D), jnp.float32),
        mesh=mesh,
        compiler_params=pltpu.CompilerParams(use_tc_tiling_on_sc=True),
        scratch_types=[
            pltpu.VMEM((n_per_tile,), jnp.int32),
            pltpu.VMEM((n_per_tile,), jnp.int32),
            pltpu.VMEM((CHUNK, DH), jnp.float32),
            pltpu.VMEM((CHUNK, DH), jnp.float32),
            pltpu.VMEM((CHUNK, DH), jnp.float32),
            pltpu.VMEM((CHUNK, DH), jnp.float32),
            pltpu.VMEM((CHUNK, D), jnp.float32),
            pltpu.VMEM((CHUNK, D), jnp.float32),
            pltpu.SemaphoreType.DMA,
            pltpu.SemaphoreType.DMA,
            pltpu.SemaphoreType.DMA,
            pltpu.SemaphoreType.DMA,
            pltpu.SemaphoreType.DMA,
            pltpu.SemaphoreType.DMA,
            pltpu.SemaphoreType.DMA,
            pltpu.SemaphoreType.DMA,
        ],
    )
    out = run(widx, pidx, wa, wb, pos_emb)
    return out.reshape(B, S, D)


# speedup vs baseline: 1.9351x; 1.9351x over previous
import jax
import jax.numpy as jnp
from jax import lax
from jax.experimental import pallas as pl
from jax.experimental.pallas import tpu as pltpu
from jax.experimental.pallas import tpu_sc as plsc

NC = 2
NS = 16
LANES = 16

CHUNK = 128
D = 64


def _sc_kernel_body(widx_hbm, pidx_hbm, word_hbm, pos_hbm, out_hbm,
                    wi_v, pi_v, wbuf0, wbuf1, pbuf0, pbuf1,
                    semw0, semw1, semp0, semp1, sems0, sems1):
    wid = lax.axis_index("c") * NS + lax.axis_index("s")
    n_per_tile = widx_hbm.shape[0] // (NC * NS)
    n_chunks = n_per_tile // CHUNK
    base = wid * n_per_tile

    pltpu.sync_copy(widx_hbm.at[pl.ds(base, n_per_tile)], wi_v)
    pltpu.sync_copy(pidx_hbm.at[pl.ds(base, n_per_tile)], pi_v)

    wbufs = [wbuf0, wbuf1]
    pbufs = [pbuf0, pbuf1]
    semws = [semw0, semw1]
    semps = [semp0, semp1]
    semss = [sems0, sems1]
    cs = [None, None]

    def issue_chunk(k, b):
        koff = k * CHUNK
        wbuf, pbuf = wbufs[b], pbufs[b]
        semw, semp = semws[b], semps[b]

        def issue_body(g, _):
            iw = wi_v[pl.ds(koff + g * LANES, LANES)]
            ip = pi_v[pl.ds(koff + g * LANES, LANES)]
            for j in range(LANES):
                r = g * LANES + j
                pltpu.async_copy(word_hbm.at[iw[j]], wbuf.at[r], semw)
                pltpu.async_copy(pos_hbm.at[ip[j]], pbuf.at[r], semp)
            return 0

        lax.fori_loop(0, CHUNK // LANES, issue_body, 0, unroll=False)

    def drain_chunk(b):
        wbuf, pbuf = wbufs[b], pbufs[b]
        semw, semp = semws[b], semps[b]

        def drain_body(r, _):
            pltpu.make_async_copy(word_hbm.at[0], wbuf.at[r], semw).wait()
            pltpu.make_async_copy(pos_hbm.at[0], pbuf.at[r], semp).wait()
            return 0

        lax.fori_loop(0, CHUNK, drain_body, 0, unroll=False)

    issue_chunk(0, 0)

    for k in range(n_chunks):
        b = k % 2
        nb = (k + 1) % 2
        if k + 1 < n_chunks:
            if cs[nb] is not None:
                cs[nb].wait()
            issue_chunk(k + 1, nb)
        drain_chunk(b)

        wbuf, pbuf = wbufs[b], pbufs[b]

        def add_body(r, _):
            for c in range(D // LANES):
                sl = pl.ds(c * LANES, LANES)
                wbuf[r, sl] = wbuf[r, sl] + pbuf[r, sl]
            return 0

        lax.fori_loop(0, CHUNK, add_body, 0, unroll=False)
        cs[b] = pltpu.async_copy(
            wbuf, out_hbm.at[pl.ds(base + k * CHUNK, CHUNK)], semss[b])

    for c in cs:
        if c is not None:
            c.wait()


def kernel(src_word, src_pos, word_emb, pos_emb):
    B, S = src_word.shape
    n = B * S
    widx = src_word.reshape(n)
    pidx = src_pos.reshape(n)

    mesh = plsc.VectorSubcoreMesh(core_axis_name="c", subcore_axis_name="s",
                                  num_cores=NC, num_subcores=NS)
    n_per_tile = n // (NC * NS)
    run = pl.kernel(
        _sc_kernel_body,
        out_type=jax.ShapeDtypeStruct((n, D), jnp.float32),
        mesh=mesh,
        compiler_params=pltpu.CompilerParams(use_tc_tiling_on_sc=True),
        scratch_types=[
            pltpu.VMEM((n_per_tile,), jnp.int32),
            pltpu.VMEM((n_per_tile,), jnp.int32),
            pltpu.VMEM((CHUNK, D), jnp.float32),
            pltpu.VMEM((CHUNK, D), jnp.float32),
            pltpu.VMEM((CHUNK, D), jnp.float32),
            pltpu.VMEM((CHUNK, D), jnp.float32),
            pltpu.SemaphoreType.DMA,
            pltpu.SemaphoreType.DMA,
            pltpu.SemaphoreType.DMA,
            pltpu.SemaphoreType.DMA,
            pltpu.SemaphoreType.DMA,
            pltpu.SemaphoreType.DMA,
        ],
    )
    out = run(widx, pidx, word_emb, pos_emb)
    return out.reshape(B, S, D)
